# Initial kernel scaffold; baseline (speedup 1.0000x reference)
import functools

import jax
import jax.numpy as jnp
from jax import lax
from jax.experimental import pallas as pl
from jax.experimental.pallas import tpu as pltpu
from jax.experimental.pallas import tpu_sc as plsc

_N = 10000
_D = 128
_H = 128
_O = 64
_E = 160000
_HALF = _H // 2
_NS = 16
_EPT = _E // _NS
_K = 80
_NCH = _EPT // _K
_RPT = _N // _NS
_ZROWS = 125


def _proj_body(x_ref, w_ref, o_ref):
    o_ref[...] = jnp.dot(x_ref[...], w_ref[...],
                         preferred_element_type=jnp.float32)


def _project(x, w):
    nb = 10
    blk = _N // nb
    return pl.pallas_call(
        _proj_body,
        grid=(2, nb),
        in_specs=[
            pl.BlockSpec((blk, _D), lambda c, b: (b, 0)),
            pl.BlockSpec((_D, _HALF), lambda c, b: (0, c)),
        ],
        out_specs=pl.BlockSpec((blk, _HALF), lambda c, b: (c * nb + b, 0)),
        out_shape=jax.ShapeDtypeStruct((2 * _N, _HALF), jnp.float32),
    )(x, w)


def _spmm_body(xq, xi, xt,
               qi_row_f, qi_col_f, it_row_f, it_col_f,
               qi_row_2, qi_col_2, it_row_2, it_col_2,
               qi_val_f, it_val_f,
               out_q, out_i, out_t,
               src_v, dst_v, val_v, rows_v, z_v,
               acc_q, acc_i, acc_t, sem):
    c = lax.axis_index("c")
    s = lax.axis_index("s")
    roff = c * _N
    zv16 = jnp.zeros((16,), jnp.float32)

    def _zrow(r, carry):
        for q in range(_HALF // 16):
            z_v[r, pl.ds(q * 16, 16)] = zv16
        return carry
    lax.fori_loop(0, _ZROWS, _zrow, 0)
    for acc in (acc_q, acc_i, acc_t):
        for k in range(_RPT // _ZROWS):
            pltpu.sync_copy(z_v, acc.at[pl.ds(s * _RPT + k * _ZROWS, _ZROWS)])
    plsc.subcore_barrier()

    relations = (
        (xi, qi_col_f, qi_row_2, qi_val_f, acc_q),
        (xq, qi_row_f, qi_col_2, qi_val_f, acc_i),
        (xt, it_col_f, it_row_2, it_val_f, acc_i),
        (xi, it_row_f, it_col_2, it_val_f, acc_t),
    )
    ebase = s * _EPT
    for tab, srcf, dst2, valf, acc in relations:
        pltpu.sync_copy(srcf.at[pl.ds(ebase, _EPT)], src_v)
        pltpu.sync_copy(dst2.at[pl.ds(s * _NCH, _NCH)], dst_v)
        pltpu.sync_copy(valf.at[pl.ds(ebase, _EPT)], val_v)

        def _off(k, carry):
            src_v[pl.ds(k * 16, 16)] = src_v[pl.ds(k * 16, 16)] + roff
            return carry
        lax.fori_loop(0, _EPT // 16, _off, 0)

        def _chunk(j, carry, tab=tab, acc=acc):
            cp = pltpu.async_copy(tab.at[src_v.at[pl.ds(j * _K, _K)]],
                                  rows_v, sem)
            cp.wait()

            def _scale(e, cc):
                v = plsc.load_gather(
                    val_v, [jnp.full((16,), j * _K + e, jnp.int32)])
                for q in range(_HALF // 16):
                    rows_v[e, pl.ds(q * 16, 16)] = (
                        rows_v[e, pl.ds(q * 16, 16)] * v)
                return cc
            lax.fori_loop(0, _K, _scale, 0)
            pltpu.sync_copy(rows_v, acc.at[dst_v.at[j]], add=True)
            return carry
        lax.fori_loop(0, _NCH, _chunk, 0)

    plsc.subcore_barrier()

    for acc, out in ((acc_q, out_q), (acc_i, out_i), (acc_t, out_t)):
        for k in range(_RPT // _ZROWS):
            rbase = s * _RPT + k * _ZROWS
            pltpu.sync_copy(acc.at[pl.ds(rbase, _ZROWS)], z_v)
            pltpu.sync_copy(z_v, out.at[pl.ds(c * _N + rbase, _ZROWS)])


_spmm = functools.partial(
    pl.kernel,
    out_type=[jax.ShapeDtypeStruct((2 * _N, _HALF), jnp.float32)] * 3,
    mesh=plsc.VectorSubcoreMesh(core_axis_name="c", subcore_axis_name="s"),
    scratch_types=[
        pltpu.VMEM((_EPT,), jnp.int32),
        pltpu.VMEM((_NCH, _K), jnp.int32),
        pltpu.VMEM((_EPT,), jnp.float32),
        pltpu.VMEM((_K, _HALF), jnp.float32),
        pltpu.VMEM((_ZROWS, _HALF), jnp.float32),
        pltpu.VMEM_SHARED((_N, _HALF), jnp.float32),
        pltpu.VMEM_SHARED((_N, _HALF), jnp.float32),
        pltpu.VMEM_SHARED((_N, _HALF), jnp.float32),
        pltpu.SemaphoreType.DMA,
    ],
)(_spmm_body)


def _embed_body(accq, acci, acct, fq, fi, ft,
                wq, bq, gq, betaq, wi, bi, gi, betai, wt, bt, gt, betat,
                qmat, aq_o, ai_o, ei_o, et_o):
    def emb(acc_ref, f_ref, w_ref, b_ref, g_ref, beta_ref):
        h0 = jnp.maximum(acc_ref[0:_N, :], 0.0)
        h1 = jnp.maximum(acc_ref[_N:2 * _N, :], 0.0)
        z = (jnp.dot(h0, w_ref[0:_HALF, :],
                     preferred_element_type=jnp.float32)
             + jnp.dot(h1, w_ref[_HALF:_H, :],
                       preferred_element_type=jnp.float32)
             + jnp.dot(f_ref[...], w_ref[_H:_H + _D, :],
                       preferred_element_type=jnp.float32)
             + b_ref[...])
        m = jnp.mean(z, axis=0, keepdims=True)
        v = jnp.mean((z - m) ** 2, axis=0, keepdims=True)
        zn = g_ref[...] * (z - m) / jnp.sqrt(v + 1e-5) + beta_ref[...]
        return jnp.maximum(zn, 0.0)

    eq = emb(accq, fq, wq, bq, gq, betaq)
    ei = emb(acci, fi, wi, bi, gi, betai)
    et = emb(acct, ft, wt, bt, gt, betat)
    q = qmat[...]
    aq_o[...] = jnp.dot(eq, q, preferred_element_type=jnp.float32)
    ai_o[...] = jnp.dot(ei, q, preferred_element_type=jnp.float32)
    ei_o[...] = ei
    et_o[...] = et


def _embed(accq, acci, acct, fq, fi, ft, params):
    return pl.pallas_call(
        _embed_body,
        out_shape=[jax.ShapeDtypeStruct((_N, _O), jnp.float32)] * 4,
    )(accq, acci, acct, fq, fi, ft, *params)


def _score_body(aq, ei, ai, et, oqi, oit):
    nt = (((1,), (1,)), ((), ()))
    oqi[...] = lax.dot_general(aq[...], ei[...], nt,
                               preferred_element_type=jnp.float32)
    oit[...] = lax.dot_general(ai[...], et[...], nt,
                               preferred_element_type=jnp.float32)


def _scores(aq, ei, ai, et):
    nb = 40
    blk = _N // nb
    return pl.pallas_call(
        _score_body,
        grid=(nb,),
        in_specs=[
            pl.BlockSpec((blk, _O), lambda b: (b, 0)),
            pl.BlockSpec((_N, _O), lambda b: (0, 0)),
            pl.BlockSpec((blk, _O), lambda b: (b, 0)),
            pl.BlockSpec((_N, _O), lambda b: (0, 0)),
        ],
        out_specs=[
            pl.BlockSpec((blk, _N), lambda b: (b, 0)),
            pl.BlockSpec((blk, _N), lambda b: (b, 0)),
        ],
        out_shape=[jax.ShapeDtypeStruct((_N, _N), jnp.float32)] * 2,
    )(aq, ei, ai, et)


def kernel(feature_q, feature_i, feature_t, qi_row, qi_col, qi_val,
           it_row, it_col, it_val, W, W_q, b_q, g_q, beta_q,
           W_i, b_i, g_i, beta_i, W_t, b_t, g_t, beta_t, Q):
    xq = _project(feature_q, W)
    xi = _project(feature_i, W)
    xt = _project(feature_t, W)

    i32 = jnp.int32
    acc_q, acc_i, acc_t = _spmm(
        xq, xi, xt,
        qi_row.astype(i32), qi_col.astype(i32),
        it_row.astype(i32), it_col.astype(i32),
        qi_row.astype(i32).reshape(_E // _K, _K),
        qi_col.astype(i32).reshape(_E // _K, _K),
        it_row.astype(i32).reshape(_E // _K, _K),
        it_col.astype(i32).reshape(_E // _K, _K),
        qi_val, it_val,
    )

    params = (W_q, b_q.reshape(1, _O), g_q.reshape(1, _O),
              beta_q.reshape(1, _O),
              W_i, b_i.reshape(1, _O), g_i.reshape(1, _O),
              beta_i.reshape(1, _O),
              W_t, b_t.reshape(1, _O), g_t.reshape(1, _O),
              beta_t.reshape(1, _O), Q)
    aq, ai, ei, et = _embed(acc_q, acc_i, acc_t,
                            feature_q, feature_i, feature_t, params)
    score_qi, score_it = _scores(aq, ei, ai, et)
    return (score_qi, score_it)


# speedup vs baseline: 2.3636x; 2.3636x over previous
import functools

import jax
import jax.numpy as jnp
from jax import lax
from jax.experimental import pallas as pl
from jax.experimental.pallas import tpu as pltpu
from jax.experimental.pallas import tpu_sc as plsc

_N = 10000
_D = 128
_H = 128
_O = 64
_E = 160000
_HALF = _H // 2
_NS = 16
_EPT = _E // _NS
_K = 80
_NCH = _EPT // _K
_ZROWS = 400
_ZCH = _N // _ZROWS


def _proj_body(x_ref, w_ref, o_ref):
    o_ref[...] = jnp.dot(x_ref[...], w_ref[0],
                         preferred_element_type=jnp.float32)


def _project(x, w_split):
    nb = 10
    blk = _N // nb
    return pl.pallas_call(
        _proj_body,
        grid=(2, nb),
        in_specs=[
            pl.BlockSpec((blk, _D), lambda c, b: (b, 0)),
            pl.BlockSpec((1, _D, _HALF), lambda c, b: (c, 0, 0)),
        ],
        out_specs=pl.BlockSpec((blk, _HALF), lambda c, b: (c * nb + b, 0)),
        out_shape=jax.ShapeDtypeStruct((2 * _N, _HALF), jnp.float32),
    )(x, w_split)


def _spmm_body(xq, xi, xt,
               qi_row_f, qi_col_f, it_row_f, it_col_f,
               qi_val_f, it_val_f,
               out_q, out_i, out_t,
               src_v, dst_v, didx_v, val_v, rows_v, z_v,
               acc_a, sem):
    c = lax.axis_index("c")
    s = lax.axis_index("s")
    roff = c * _N
    zv16 = jnp.zeros((16,), jnp.float32)
    ebase = s * _EPT

    def fill_zeros():
        def _zrow(r, carry):
            for q in range(_HALF // 16):
                z_v[r, pl.ds(q * 16, 16)] = zv16
            return carry
        lax.fori_loop(0, _ZROWS, _zrow, 0)

    def zero_acc(acc):
        for k in range(2):
            cid = s + _NS * k

            @pl.when(cid < _ZCH)
            def _():
                pltpu.sync_copy(z_v, acc.at[pl.ds(cid * _ZROWS, _ZROWS)])

    def run_relation(tab, srcf, dstf, valf, acc):
        pltpu.sync_copy(srcf.at[pl.ds(ebase, _EPT)], src_v)
        pltpu.sync_copy(dstf.at[pl.ds(ebase, _EPT)], dst_v)
        pltpu.sync_copy(valf.at[pl.ds(ebase, _EPT)], val_v)

        def _off(k, carry):
            src_v[pl.ds(k * 16, 16)] = src_v[pl.ds(k * 16, 16)] + roff
            return carry
        lax.fori_loop(0, _EPT // 16, _off, 0)

        def _chunk(j, carry):
            cp = pltpu.async_copy(tab.at[src_v.at[pl.ds(j * _K, _K)]],
                                  rows_v, sem)
            cp.wait()

            def _scale(e, cc):
                v = plsc.load_gather(
                    val_v, [jnp.full((16,), j * _K + e, jnp.int32)])
                for q in range(_HALF // 16):
                    rows_v[e, pl.ds(q * 16, 16)] = (
                        rows_v[e, pl.ds(q * 16, 16)] * v)
                return cc
            lax.fori_loop(0, _K, _scale, 0)
            for g in range(_K // 16):
                didx_v[pl.ds(g * 16, 16)] = dst_v[pl.ds(j * _K + g * 16, 16)]
            pltpu.sync_copy(rows_v, acc.at[didx_v], add=True)
            return carry
        lax.fori_loop(0, _NCH, _chunk, 0)

    def write_out(acc, out):
        for k in range(2):
            cid = s + _NS * k

            @pl.when(cid < _ZCH)
            def _():
                rbase = cid * _ZROWS
                pltpu.sync_copy(acc.at[pl.ds(rbase, _ZROWS)], z_v)
                pltpu.sync_copy(z_v, out.at[pl.ds(c * _N + rbase, _ZROWS)])

    fill_zeros()
    zero_acc(acc_a)
    plsc.subcore_barrier()
    run_relation(xi, qi_col_f, qi_row_f, qi_val_f, acc_a)
    plsc.subcore_barrier()
    write_out(acc_a, out_q)
    fill_zeros()
    zero_acc(acc_a)
    plsc.subcore_barrier()
    run_relation(xq, qi_row_f, qi_col_f, qi_val_f, acc_a)
    run_relation(xt, it_col_f, it_row_f, it_val_f, acc_a)
    plsc.subcore_barrier()
    write_out(acc_a, out_i)
    fill_zeros()
    zero_acc(acc_a)
    plsc.subcore_barrier()
    run_relation(xi, it_row_f, it_col_f, it_val_f, acc_a)
    plsc.subcore_barrier()
    write_out(acc_a, out_t)


_spmm = functools.partial(
    pl.kernel,
    out_type=[jax.ShapeDtypeStruct((2 * _N, _HALF), jnp.float32)] * 3,
    mesh=plsc.VectorSubcoreMesh(core_axis_name="c", subcore_axis_name="s"),
    compiler_params=pltpu.CompilerParams(needs_layout_passes=False,
                                         use_tc_tiling_on_sc=False),
    scratch_types=[
        pltpu.VMEM((_EPT,), jnp.int32),
        pltpu.VMEM((_EPT,), jnp.int32),
        pltpu.VMEM((_K,), jnp.int32),
        pltpu.VMEM((_EPT,), jnp.float32),
        pltpu.VMEM((_K, _HALF), jnp.float32),
        pltpu.VMEM((_ZROWS, _HALF), jnp.float32),
        pltpu.VMEM_SHARED((_N, _HALF), jnp.float32),
        pltpu.SemaphoreType.DMA,
    ],
)(_spmm_body)


def _embed_body(acc_ref, f_ref, w_ref, b_ref, g_ref, beta_ref, qmat,
                e_o, a_o):
    h0 = jnp.maximum(acc_ref[0:_N, :], 0.0)
    h1 = jnp.maximum(acc_ref[_N:2 * _N, :], 0.0)
    z = (jnp.dot(h0, w_ref[0:_HALF, :],
                 preferred_element_type=jnp.float32)
         + jnp.dot(h1, w_ref[_HALF:_H, :],
                   preferred_element_type=jnp.float32)
         + jnp.dot(f_ref[...], w_ref[_H:_H + _D, :],
                   preferred_element_type=jnp.float32)
         + b_ref[...])
    m = jnp.mean(z, axis=0, keepdims=True)
    v = jnp.mean((z - m) ** 2, axis=0, keepdims=True)
    zn = g_ref[...] * (z - m) / jnp.sqrt(v + 1e-5) + beta_ref[...]
    e = jnp.maximum(zn, 0.0)
    e_o[...] = e
    a_o[...] = jnp.dot(e, qmat[...], preferred_element_type=jnp.float32)


def _embed(acc, f, w, b, g, beta, qmat):
    return pl.pallas_call(
        _embed_body,
        out_shape=[jax.ShapeDtypeStruct((_N, _O), jnp.float32)] * 2,
    )(acc, f, w, b.reshape(1, _O), g.reshape(1, _O), beta.reshape(1, _O),
      qmat)


def _score_body(aq, ei, ai, et, oqi, oit):
    nt = (((1,), (1,)), ((), ()))
    oqi[...] = lax.dot_general(aq[...], ei[...], nt,
                               preferred_element_type=jnp.float32)
    oit[...] = lax.dot_general(ai[...], et[...], nt,
                               preferred_element_type=jnp.float32)


def _scores(aq, ei, ai, et):
    nb = 50
    blk = _N // nb
    return pl.pallas_call(
        _score_body,
        grid=(nb,),
        in_specs=[
            pl.BlockSpec((blk, _O), lambda b: (b, 0)),
            pl.BlockSpec((_N, _O), lambda b: (0, 0)),
            pl.BlockSpec((blk, _O), lambda b: (b, 0)),
            pl.BlockSpec((_N, _O), lambda b: (0, 0)),
        ],
        out_specs=[
            pl.BlockSpec((blk, _N), lambda b: (b, 0)),
            pl.BlockSpec((blk, _N), lambda b: (b, 0)),
        ],
        out_shape=[jax.ShapeDtypeStruct((_N, _N), jnp.float32)] * 2,
    )(aq, ei, ai, et)


def kernel(feature_q, feature_i, feature_t, qi_row, qi_col, qi_val,
           it_row, it_col, it_val, W, W_q, b_q, g_q, beta_q,
           W_i, b_i, g_i, beta_i, W_t, b_t, g_t, beta_t, Q):
    w_split = W.reshape(_D, 2, _HALF).transpose(1, 0, 2)
    xq = _project(feature_q, w_split)
    xi = _project(feature_i, w_split)
    xt = _project(feature_t, w_split)

    i32 = jnp.int32
    acc_q, acc_i, acc_t = _spmm(
        xq, xi, xt,
        qi_row.astype(i32), qi_col.astype(i32),
        it_row.astype(i32), it_col.astype(i32),
        qi_val, it_val,
    )

    _, aq = _embed(acc_q, feature_q, W_q, b_q, g_q, beta_q, Q)
    ei, ai = _embed(acc_i, feature_i, W_i, b_i, g_i, beta_i, Q)
    et, _ = _embed(acc_t, feature_t, W_t, b_t, g_t, beta_t, Q)
    score_qi, score_it = _scores(aq, ei, ai, et)
    return (score_qi, score_it)


# speedup vs baseline: 3.4965x; 1.4793x over previous
import functools

import jax
import jax.numpy as jnp
from jax import lax
from jax.experimental import pallas as pl
from jax.experimental.pallas import tpu as pltpu
from jax.experimental.pallas import tpu_sc as plsc

_N = 10000
_D = 128
_H = 128
_O = 64
_E = 160000
_HALF = _H // 2
_NS = 16
_EPT = _E // _NS
_K = 80
_NCH = _EPT // _K
_ZROWS = 400
_ZCH = _N // _ZROWS


def _proj_body(x_ref, w_ref, o_ref):
    o_ref[...] = jnp.dot(x_ref[...], w_ref[0],
                         preferred_element_type=jnp.float32)


def _project(x, w_split):
    nb = 10
    blk = _N // nb
    return pl.pallas_call(
        _proj_body,
        grid=(2, nb),
        in_specs=[
            pl.BlockSpec((blk, _D), lambda c, b: (b, 0)),
            pl.BlockSpec((1, _D, _HALF), lambda c, b: (c, 0, 0)),
        ],
        out_specs=pl.BlockSpec((blk, _HALF), lambda c, b: (c * nb + b, 0)),
        out_shape=jax.ShapeDtypeStruct((2 * _N, _HALF), jnp.float32),
    )(x, w_split)


def _spmm_body(xq, xi, xt,
               qi_row_f, qi_col_f, it_row_f, it_col_f,
               qi_val_f, it_val_f,
               out_q, out_i, out_t,
               src_v, dst_v, didx_v, val_v, rows_a, rows_b, z_v,
               acc_a, sem_a, sem_b):
    c = lax.axis_index("c")
    s = lax.axis_index("s")
    roff = c * _N
    zv16 = jnp.zeros((16,), jnp.float32)
    ebase = s * _EPT

    def fill_zeros():
        def _zrow(r, carry):
            for q in range(_HALF // 16):
                z_v[r, pl.ds(q * 16, 16)] = zv16
            return carry
        lax.fori_loop(0, _ZROWS, _zrow, 0)

    def zero_acc(acc):
        for k in range(2):
            cid = s + _NS * k

            @pl.when(cid < _ZCH)
            def _():
                pltpu.sync_copy(z_v, acc.at[pl.ds(cid * _ZROWS, _ZROWS)])

    def run_relation(tab, srcf, dstf, valf, acc):
        pltpu.sync_copy(srcf.at[pl.ds(ebase, _EPT)], src_v)
        pltpu.sync_copy(dstf.at[pl.ds(ebase, _EPT)], dst_v)
        pltpu.sync_copy(valf.at[pl.ds(ebase, _EPT)], val_v)

        def _off(k, carry):
            src_v[pl.ds(k * 16, 16)] = src_v[pl.ds(k * 16, 16)] + roff
            return carry
        lax.fori_loop(0, _EPT // 16, _off, 0)

        def gather_start(j, buf, sem):
            pltpu.async_copy(tab.at[src_v.at[pl.ds(j * _K, _K)]], buf, sem)

        def gather_wait(j, buf, sem):
            pltpu.make_async_copy(tab.at[src_v.at[pl.ds(j * _K, _K)]],
                                  buf, sem).wait()

        def scale_scatter(j, buf):
            def _scale(e2, cc):
                for u in range(2):
                    e = e2 * 2 + u
                    v = plsc.load_gather(
                        val_v, [jnp.full((16,), j * _K + e, jnp.int32)])
                    for q in range(_HALF // 16):
                        buf[e, pl.ds(q * 16, 16)] = (
                            buf[e, pl.ds(q * 16, 16)] * v)
                return cc
            lax.fori_loop(0, _K // 2, _scale, 0)
            for g in range(_K // 16):
                didx_v[pl.ds(g * 16, 16)] = dst_v[pl.ds(j * _K + g * 16, 16)]
            pltpu.sync_copy(buf, acc.at[didx_v], add=True)

        gather_start(0, rows_a, sem_a)

        def _pair(j2, carry):
            j = j2 * 2
            gather_start(j + 1, rows_b, sem_b)
            gather_wait(j, rows_a, sem_a)
            scale_scatter(j, rows_a)
            gather_start(j + 2, rows_a, sem_a)
            gather_wait(j + 1, rows_b, sem_b)
            scale_scatter(j + 1, rows_b)
            return carry
        lax.fori_loop(0, (_NCH - 1) // 2, _pair, 0)
        gather_wait(_NCH - 1, rows_a, sem_a)
        scale_scatter(_NCH - 1, rows_a)

    def write_out(acc, out):
        for k in range(2):
            cid = s + _NS * k

            @pl.when(cid < _ZCH)
            def _():
                rbase = cid * _ZROWS
                pltpu.sync_copy(acc.at[pl.ds(rbase, _ZROWS)], z_v)
                pltpu.sync_copy(z_v, out.at[pl.ds(c * _N + rbase, _ZROWS)])

    fill_zeros()
    zero_acc(acc_a)
    plsc.subcore_barrier()
    run_relation(xi, qi_col_f, qi_row_f, qi_val_f, acc_a)
    plsc.subcore_barrier()
    write_out(acc_a, out_q)
    fill_zeros()
    zero_acc(acc_a)
    plsc.subcore_barrier()
    run_relation(xq, qi_row_f, qi_col_f, qi_val_f, acc_a)
    run_relation(xt, it_col_f, it_row_f, it_val_f, acc_a)
    plsc.subcore_barrier()
    write_out(acc_a, out_i)
    fill_zeros()
    zero_acc(acc_a)
    plsc.subcore_barrier()
    run_relation(xi, it_row_f, it_col_f, it_val_f, acc_a)
    plsc.subcore_barrier()
    write_out(acc_a, out_t)


_spmm = functools.partial(
    pl.kernel,
    out_type=[jax.ShapeDtypeStruct((2 * _N, _HALF), jnp.float32)] * 3,
    mesh=plsc.VectorSubcoreMesh(core_axis_name="c", subcore_axis_name="s"),
    compiler_params=pltpu.CompilerParams(needs_layout_passes=False,
                                         use_tc_tiling_on_sc=False),
    scratch_types=[
        pltpu.VMEM((_EPT,), jnp.int32),
        pltpu.VMEM((_EPT,), jnp.int32),
        pltpu.VMEM((_K,), jnp.int32),
        pltpu.VMEM((_EPT,), jnp.float32),
        pltpu.VMEM((_K, _HALF), jnp.float32),
        pltpu.VMEM((_K, _HALF), jnp.float32),
        pltpu.VMEM((_ZROWS, _HALF), jnp.float32),
        pltpu.VMEM_SHARED((_N, _HALF), jnp.float32),
        pltpu.SemaphoreType.DMA,
        pltpu.SemaphoreType.DMA,
    ],
)(_spmm_body)


def _embed_body(acc_ref, f_ref, w_ref, b_ref, g_ref, beta_ref, qmat,
                e_o, a_o):
    h0 = jnp.maximum(acc_ref[0:_N, :], 0.0)
    h1 = jnp.maximum(acc_ref[_N:2 * _N, :], 0.0)
    z = (jnp.dot(h0, w_ref[0:_HALF, :],
                 preferred_element_type=jnp.float32)
         + jnp.dot(h1, w_ref[_HALF:_H, :],
                   preferred_element_type=jnp.float32)
         + jnp.dot(f_ref[...], w_ref[_H:_H + _D, :],
                   preferred_element_type=jnp.float32)
         + b_ref[...])
    m = jnp.mean(z, axis=0, keepdims=True)
    v = jnp.mean((z - m) ** 2, axis=0, keepdims=True)
    zn = g_ref[...] * (z - m) / jnp.sqrt(v + 1e-5) + beta_ref[...]
    e = jnp.maximum(zn, 0.0)
    e_o[...] = e
    a_o[...] = jnp.dot(e, qmat[...], preferred_element_type=jnp.float32)


def _embed(acc, f, w, b, g, beta, qmat):
    return pl.pallas_call(
        _embed_body,
        out_shape=[jax.ShapeDtypeStruct((_N, _O), jnp.float32)] * 2,
    )(acc, f, w, b.reshape(1, _O), g.reshape(1, _O), beta.reshape(1, _O),
      qmat)


def _score_body(aq, ei, ai, et, oqi, oit):
    nt = (((1,), (1,)), ((), ()))
    oqi[...] = lax.dot_general(aq[...], ei[...], nt,
                               preferred_element_type=jnp.float32)
    oit[...] = lax.dot_general(ai[...], et[...], nt,
                               preferred_element_type=jnp.float32)


def _scores(aq, ei, ai, et):
    nb = 50
    blk = _N // nb
    return pl.pallas_call(
        _score_body,
        grid=(nb,),
        in_specs=[
            pl.BlockSpec((blk, _O), lambda b: (b, 0)),
            pl.BlockSpec((_N, _O), lambda b: (0, 0)),
            pl.BlockSpec((blk, _O), lambda b: (b, 0)),
            pl.BlockSpec((_N, _O), lambda b: (0, 0)),
        ],
        out_specs=[
            pl.BlockSpec((blk, _N), lambda b: (b, 0)),
            pl.BlockSpec((blk, _N), lambda b: (b, 0)),
        ],
        out_shape=[jax.ShapeDtypeStruct((_N, _N), jnp.float32)] * 2,
    )(aq, ei, ai, et)


def kernel(feature_q, feature_i, feature_t, qi_row, qi_col, qi_val,
           it_row, it_col, it_val, W, W_q, b_q, g_q, beta_q,
           W_i, b_i, g_i, beta_i, W_t, b_t, g_t, beta_t, Q):
    w_split = W.reshape(_D, 2, _HALF).transpose(1, 0, 2)
    xq = _project(feature_q, w_split)
    xi = _project(feature_i, w_split)
    xt = _project(feature_t, w_split)

    i32 = jnp.int32
    acc_q, acc_i, acc_t = _spmm(
        xq, xi, xt,
        qi_row.astype(i32), qi_col.astype(i32),
        it_row.astype(i32), it_col.astype(i32),
        qi_val, it_val,
    )

    _, aq = _embed(acc_q, feature_q, W_q, b_q, g_q, beta_q, Q)
    ei, ai = _embed(acc_i, feature_i, W_i, b_i, g_i, beta_i, Q)
    et, _ = _embed(acc_t, feature_t, W_t, b_t, g_t, beta_t, Q)
    score_qi, score_it = _scores(aq, ei, ai, et)
    return (score_qi, score_it)


# speedup vs baseline: 4.0523x; 1.1590x over previous
import functools

import jax
import jax.numpy as jnp
from jax import lax
from jax.experimental import pallas as pl
from jax.experimental.pallas import tpu as pltpu
from jax.experimental.pallas import tpu_sc as plsc

_N = 10000
_D = 128
_H = 128
_O = 64
_E = 160000
_HALF = _H // 2
_NS = 16
_EPT = _E // _NS
_K = 80
_NCH = _EPT // _K
_ZROWS = 400
_ZCH = _N // _ZROWS


def _proj_body(x_ref, w_ref, o_ref):
    o_ref[...] = jnp.dot(x_ref[...], w_ref[0],
                         preferred_element_type=jnp.float32)


def _project(x, w_split):
    nb = 10
    blk = _N // nb
    return pl.pallas_call(
        _proj_body,
        grid=(2, nb),
        in_specs=[
            pl.BlockSpec((blk, _D), lambda c, b: (b, 0)),
            pl.BlockSpec((1, _D, _HALF), lambda c, b: (c, 0, 0)),
        ],
        out_specs=pl.BlockSpec((blk, _HALF), lambda c, b: (c * nb + b, 0)),
        out_shape=jax.ShapeDtypeStruct((2 * _N, _HALF), jnp.float32),
    )(x, w_split)


def _spmm_body(xq, xi, xt,
               qi_row_f, qi_col_f, it_row_f, it_col_f,
               qi_val_f, it_val_f,
               out_q, out_i, out_t,
               src_v, dst_v, didx_v, val_v, rows_v, z_v,
               acc_a, gsem, ssem):
    c = lax.axis_index("c")
    s = lax.axis_index("s")
    roff = c * _N
    zv16 = jnp.zeros((16,), jnp.float32)
    ebase = s * _EPT

    def fill_zeros():
        def _zrow(r, carry):
            for q in range(_HALF // 16):
                z_v[r, pl.ds(q * 16, 16)] = zv16
            return carry
        lax.fori_loop(0, _ZROWS, _zrow, 0)

    def zero_acc(acc):
        for k in range(2):
            cid = s + _NS * k

            @pl.when(cid < _ZCH)
            def _():
                pltpu.sync_copy(z_v, acc.at[pl.ds(cid * _ZROWS, _ZROWS)])

    def run_relation(tab, srcf, dstf, valf, acc):
        pltpu.sync_copy(srcf.at[pl.ds(ebase, _EPT)], src_v)
        pltpu.sync_copy(dstf.at[pl.ds(ebase, _EPT)], dst_v)
        pltpu.sync_copy(valf.at[pl.ds(ebase, _EPT)], val_v)

        def _off(k, carry):
            src_v[pl.ds(k * 16, 16)] = src_v[pl.ds(k * 16, 16)] + roff
            return carry
        lax.fori_loop(0, _EPT // 16, _off, 0)

        def gs(j, r):
            pltpu.async_copy(tab.at[src_v.at[pl.ds(j * _K, _K)]],
                             rows_v.at[r], gsem[r])

        def gw(j, r):
            pltpu.make_async_copy(tab.at[src_v.at[pl.ds(j * _K, _K)]],
                                  rows_v.at[r], gsem[r]).wait()

        def ss(j, r):
            pltpu.async_copy(rows_v.at[r], acc.at[didx_v.at[r]],
                             ssem[r], add=True)

        def sw(r):
            pltpu.make_async_copy(rows_v.at[r], acc.at[didx_v.at[r]],
                                  ssem[r]).wait()

        def scale_scatter(j, r):
            def _scale(e2, cc):
                for u in range(2):
                    e = e2 * 2 + u
                    v = plsc.load_gather(
                        val_v, [jnp.full((16,), j * _K + e, jnp.int32)])
                    for q in range(_HALF // 16):
                        rows_v[r, e, pl.ds(q * 16, 16)] = (
                            rows_v[r, e, pl.ds(q * 16, 16)] * v)
                return cc
            lax.fori_loop(0, _K // 2, _scale, 0)
            for g in range(_K // 16):
                didx_v[r, pl.ds(g * 16, 16)] = (
                    dst_v[pl.ds(j * _K + g * 16, 16)])
            ss(j, r)

        gs(0, 0)
        gs(1, 1)
        gs(2, 2)
        gw(0, 0)
        scale_scatter(0, 0)
        gs(3, 3)
        gw(1, 1)
        scale_scatter(1, 1)

        def _quad(t, carry):
            j0 = 2 + 4 * t
            for u in range(4):
                j = j0 + u
                r = (2 + u) % 4
                sw((r + 2) % 4)
                gs(j + 2, (r + 2) % 4)
                gw(j, r)
                scale_scatter(j, r)
            return carry
        lax.fori_loop(0, (_NCH - 5) // 4, _quad, 0)
        sw(0)
        gs(_NCH - 1, 0)
        gw(_NCH - 3, 2)
        scale_scatter(_NCH - 3, 2)
        sw(1)
        gw(_NCH - 2, 3)
        scale_scatter(_NCH - 2, 3)
        sw(2)
        gw(_NCH - 1, 0)
        scale_scatter(_NCH - 1, 0)
        sw(3)
        sw(0)

    def write_out(acc, out):
        for k in range(2):
            cid = s + _NS * k

            @pl.when(cid < _ZCH)
            def _():
                rbase = cid * _ZROWS
                pltpu.sync_copy(acc.at[pl.ds(rbase, _ZROWS)], z_v)
                pltpu.sync_copy(z_v, out.at[pl.ds(c * _N + rbase, _ZROWS)])

    fill_zeros()
    zero_acc(acc_a)
    plsc.subcore_barrier()
    run_relation(xi, qi_col_f, qi_row_f, qi_val_f, acc_a)
    plsc.subcore_barrier()
    write_out(acc_a, out_q)
    fill_zeros()
    zero_acc(acc_a)
    plsc.subcore_barrier()
    run_relation(xq, qi_row_f, qi_col_f, qi_val_f, acc_a)
    run_relation(xt, it_col_f, it_row_f, it_val_f, acc_a)
    plsc.subcore_barrier()
    write_out(acc_a, out_i)
    fill_zeros()
    zero_acc(acc_a)
    plsc.subcore_barrier()
    run_relation(xi, it_row_f, it_col_f, it_val_f, acc_a)
    plsc.subcore_barrier()
    write_out(acc_a, out_t)


_spmm = functools.partial(
    pl.kernel,
    out_type=[jax.ShapeDtypeStruct((2 * _N, _HALF), jnp.float32)] * 3,
    mesh=plsc.VectorSubcoreMesh(core_axis_name="c", subcore_axis_name="s"),
    compiler_params=pltpu.CompilerParams(needs_layout_passes=False,
                                         use_tc_tiling_on_sc=False),
    scratch_types=[
        pltpu.VMEM((_EPT,), jnp.int32),
        pltpu.VMEM((_EPT,), jnp.int32),
        pltpu.VMEM((4, _K), jnp.int32),
        pltpu.VMEM((_EPT,), jnp.float32),
        pltpu.VMEM((4, _K, _HALF), jnp.float32),
        pltpu.VMEM((_ZROWS, _HALF), jnp.float32),
        pltpu.VMEM_SHARED((_N, _HALF), jnp.float32),
        [pltpu.SemaphoreType.DMA] * 4,
        [pltpu.SemaphoreType.DMA] * 4,
    ],
)(_spmm_body)


def _embed_body(acc_ref, f_ref, w_ref, b_ref, g_ref, beta_ref, qmat,
                e_o, a_o):
    h0 = jnp.maximum(acc_ref[0:_N, :], 0.0)
    h1 = jnp.maximum(acc_ref[_N:2 * _N, :], 0.0)
    z = (jnp.dot(h0, w_ref[0:_HALF, :],
                 preferred_element_type=jnp.float32)
         + jnp.dot(h1, w_ref[_HALF:_H, :],
                   preferred_element_type=jnp.float32)
         + jnp.dot(f_ref[...], w_ref[_H:_H + _D, :],
                   preferred_element_type=jnp.float32)
         + b_ref[...])
    m = jnp.mean(z, axis=0, keepdims=True)
    v = jnp.mean((z - m) ** 2, axis=0, keepdims=True)
    zn = g_ref[...] * (z - m) / jnp.sqrt(v + 1e-5) + beta_ref[...]
    e = jnp.maximum(zn, 0.0)
    e_o[...] = e
    a_o[...] = jnp.dot(e, qmat[...], preferred_element_type=jnp.float32)


def _embed(acc, f, w, b, g, beta, qmat):
    return pl.pallas_call(
        _embed_body,
        out_shape=[jax.ShapeDtypeStruct((_N, _O), jnp.float32)] * 2,
    )(acc, f, w, b.reshape(1, _O), g.reshape(1, _O), beta.reshape(1, _O),
      qmat)


def _score_body(aq, ei, ai, et, oqi, oit):
    nt = (((1,), (1,)), ((), ()))
    oqi[...] = lax.dot_general(aq[...], ei[...], nt,
                               preferred_element_type=jnp.float32)
    oit[...] = lax.dot_general(ai[...], et[...], nt,
                               preferred_element_type=jnp.float32)


def _scores(aq, ei, ai, et):
    nb = 50
    blk = _N // nb
    return pl.pallas_call(
        _score_body,
        grid=(nb,),
        in_specs=[
            pl.BlockSpec((blk, _O), lambda b: (b, 0)),
            pl.BlockSpec((_N, _O), lambda b: (0, 0)),
            pl.BlockSpec((blk, _O), lambda b: (b, 0)),
            pl.BlockSpec((_N, _O), lambda b: (0, 0)),
        ],
        out_specs=[
            pl.BlockSpec((blk, _N), lambda b: (b, 0)),
            pl.BlockSpec((blk, _N), lambda b: (b, 0)),
        ],
        out_shape=[jax.ShapeDtypeStruct((_N, _N), jnp.float32)] * 2,
    )(aq, ei, ai, et)


def kernel(feature_q, feature_i, feature_t, qi_row, qi_col, qi_val,
           it_row, it_col, it_val, W, W_q, b_q, g_q, beta_q,
           W_i, b_i, g_i, beta_i, W_t, b_t, g_t, beta_t, Q):
    w_split = W.reshape(_D, 2, _HALF).transpose(1, 0, 2)
    xq = _project(feature_q, w_split)
    xi = _project(feature_i, w_split)
    xt = _project(feature_t, w_split)

    i32 = jnp.int32
    acc_q, acc_i, acc_t = _spmm(
        xq, xi, xt,
        qi_row.astype(i32), qi_col.astype(i32),
        it_row.astype(i32), it_col.astype(i32),
        qi_val, it_val,
    )

    _, aq = _embed(acc_q, feature_q, W_q, b_q, g_q, beta_q, Q)
    ei, ai = _embed(acc_i, feature_i, W_i, b_i, g_i, beta_i, Q)
    et, _ = _embed(acc_t, feature_t, W_t, b_t, g_t, beta_t, Q)
    score_qi, score_it = _scores(aq, ei, ai, et)
    return (score_qi, score_it)


# speedup vs baseline: 4.0675x; 1.0037x over previous
import functools

import jax
import jax.numpy as jnp
from jax import lax
from jax.experimental import pallas as pl
from jax.experimental.pallas import tpu as pltpu
from jax.experimental.pallas import tpu_sc as plsc

_N = 10000
_D = 128
_H = 128
_O = 64
_E = 160000
_HALF = _H // 2
_NS = 16
_EPT = _E // _NS
_K = 80
_NCH = _EPT // _K
_ZROWS = 400
_ZCH = _N // _ZROWS


def _proj_body(x_ref, w_ref, o_ref):
    o_ref[...] = jnp.dot(x_ref[...], w_ref[0],
                         preferred_element_type=jnp.float32)


def _project(x, w_split):
    nb = 10
    blk = _N // nb
    return pl.pallas_call(
        _proj_body,
        grid=(2, nb),
        in_specs=[
            pl.BlockSpec((blk, _D), lambda c, b: (b, 0)),
            pl.BlockSpec((1, _D, _HALF), lambda c, b: (c, 0, 0)),
        ],
        out_specs=pl.BlockSpec((blk, _HALF), lambda c, b: (c * nb + b, 0)),
        out_shape=jax.ShapeDtypeStruct((2 * _N, _HALF), jnp.float32),
    )(x, w_split)


def _spmm_body(xq, xi, xt,
               qi_row_f, qi_col_f, it_row_f, it_col_f,
               qi_val_f, it_val_f,
               out_q, out_i, out_t,
               src_v, dst_v, didx_v, val_v, rows_v, z_v,
               acc_a, gsem, ssem):
    c = lax.axis_index("c")
    s = lax.axis_index("s")
    roff = c * _N
    zv16 = jnp.zeros((16,), jnp.float32)
    ebase = s * _EPT

    def fill_zeros():
        def _zrow(r, carry):
            for q in range(_HALF // 16):
                z_v[r, pl.ds(q * 16, 16)] = zv16
            return carry
        lax.fori_loop(0, _ZROWS, _zrow, 0)

    def zero_acc(acc):
        for k in range(2):
            cid = s + _NS * k

            @pl.when(cid < _ZCH)
            def _():
                pltpu.sync_copy(z_v, acc.at[pl.ds(cid * _ZROWS, _ZROWS)])

    def run_relation(tab, srcf, dstf, valf, acc):
        pltpu.sync_copy(srcf.at[pl.ds(ebase, _EPT)], src_v)
        pltpu.sync_copy(dstf.at[pl.ds(ebase, _EPT)], dst_v)
        pltpu.sync_copy(valf.at[pl.ds(ebase, _EPT)], val_v)

        def _off(k, carry):
            src_v[pl.ds(k * 16, 16)] = src_v[pl.ds(k * 16, 16)] + roff
            return carry
        lax.fori_loop(0, _EPT // 16, _off, 0)

        def gs(j, r):
            pltpu.async_copy(tab.at[src_v.at[pl.ds(j * _K, _K)]],
                             rows_v.at[r], gsem[r])

        def gw(j, r):
            pltpu.make_async_copy(tab.at[src_v.at[pl.ds(j * _K, _K)]],
                                  rows_v.at[r], gsem[r]).wait()

        def ss(j, r):
            pltpu.async_copy(rows_v.at[r], acc.at[didx_v.at[r]],
                             ssem[r], add=True)

        def sw(r):
            pltpu.make_async_copy(rows_v.at[r], acc.at[didx_v.at[r]],
                                  ssem[r]).wait()

        def scale_scatter(j, r):
            def _scale(e4, cc):
                for u in range(4):
                    e = e4 * 4 + u
                    v = plsc.load_gather(
                        val_v, [jnp.full((16,), j * _K + e, jnp.int32)])
                    for q in range(_HALF // 16):
                        rows_v[r, e, pl.ds(q * 16, 16)] = (
                            rows_v[r, e, pl.ds(q * 16, 16)] * v)
                return cc
            lax.fori_loop(0, _K // 4, _scale, 0)
            for g in range(_K // 16):
                didx_v[r, pl.ds(g * 16, 16)] = (
                    dst_v[pl.ds(j * _K + g * 16, 16)])
            ss(j, r)

        gs(0, 0)
        gs(1, 1)
        gs(2, 2)
        gw(0, 0)
        scale_scatter(0, 0)
        gs(3, 3)
        gw(1, 1)
        scale_scatter(1, 1)

        def _quad(t, carry):
            j0 = 2 + 4 * t
            for u in range(4):
                j = j0 + u
                r = (2 + u) % 4
                sw((r + 2) % 4)
                gs(j + 2, (r + 2) % 4)
                gw(j, r)
                scale_scatter(j, r)
            return carry
        lax.fori_loop(0, (_NCH - 5) // 4, _quad, 0)
        sw(0)
        gs(_NCH - 1, 0)
        gw(_NCH - 3, 2)
        scale_scatter(_NCH - 3, 2)
        sw(1)
        gw(_NCH - 2, 3)
        scale_scatter(_NCH - 2, 3)
        sw(2)
        gw(_NCH - 1, 0)
        scale_scatter(_NCH - 1, 0)
        sw(3)
        sw(0)

    def write_out(acc, out):
        for k in range(2):
            cid = s + _NS * k

            @pl.when(cid < _ZCH)
            def _():
                rbase = cid * _ZROWS
                pltpu.sync_copy(acc.at[pl.ds(rbase, _ZROWS)], z_v)
                pltpu.sync_copy(z_v, out.at[pl.ds(c * _N + rbase, _ZROWS)])

    fill_zeros()
    zero_acc(acc_a)
    plsc.subcore_barrier()
    run_relation(xi, qi_col_f, qi_row_f, qi_val_f, acc_a)
    plsc.subcore_barrier()
    write_out(acc_a, out_q)
    fill_zeros()
    zero_acc(acc_a)
    plsc.subcore_barrier()
    run_relation(xq, qi_row_f, qi_col_f, qi_val_f, acc_a)
    run_relation(xt, it_col_f, it_row_f, it_val_f, acc_a)
    plsc.subcore_barrier()
    write_out(acc_a, out_i)
    fill_zeros()
    zero_acc(acc_a)
    plsc.subcore_barrier()
    run_relation(xi, it_row_f, it_col_f, it_val_f, acc_a)
    plsc.subcore_barrier()
    write_out(acc_a, out_t)


_spmm = functools.partial(
    pl.kernel,
    out_type=[jax.ShapeDtypeStruct((2 * _N, _HALF), jnp.float32)] * 3,
    mesh=plsc.VectorSubcoreMesh(core_axis_name="c", subcore_axis_name="s"),
    compiler_params=pltpu.CompilerParams(needs_layout_passes=False,
                                         use_tc_tiling_on_sc=False),
    scratch_types=[
        pltpu.VMEM((_EPT,), jnp.int32),
        pltpu.VMEM((_EPT,), jnp.int32),
        pltpu.VMEM((4, _K), jnp.int32),
        pltpu.VMEM((_EPT,), jnp.float32),
        pltpu.VMEM((4, _K, _HALF), jnp.float32),
        pltpu.VMEM((_ZROWS, _HALF), jnp.float32),
        pltpu.VMEM_SHARED((_N, _HALF), jnp.float32),
        [pltpu.SemaphoreType.DMA] * 4,
        [pltpu.SemaphoreType.DMA] * 4,
    ],
)(_spmm_body)


def _embed_body(acc_ref, f_ref, w_ref, b_ref, g_ref, beta_ref, qmat,
                e_o, a_o):
    h0 = jnp.maximum(acc_ref[0:_N, :], 0.0)
    h1 = jnp.maximum(acc_ref[_N:2 * _N, :], 0.0)
    z = (jnp.dot(h0, w_ref[0:_HALF, :],
                 preferred_element_type=jnp.float32)
         + jnp.dot(h1, w_ref[_HALF:_H, :],
                   preferred_element_type=jnp.float32)
         + jnp.dot(f_ref[...], w_ref[_H:_H + _D, :],
                   preferred_element_type=jnp.float32)
         + b_ref[...])
    m = jnp.mean(z, axis=0, keepdims=True)
    v = jnp.mean((z - m) ** 2, axis=0, keepdims=True)
    zn = g_ref[...] * (z - m) / jnp.sqrt(v + 1e-5) + beta_ref[...]
    e = jnp.maximum(zn, 0.0)
    e_o[...] = e
    a_o[...] = jnp.dot(e, qmat[...], preferred_element_type=jnp.float32)


def _embed(acc, f, w, b, g, beta, qmat):
    return pl.pallas_call(
        _embed_body,
        out_shape=[jax.ShapeDtypeStruct((_N, _O), jnp.float32)] * 2,
    )(acc, f, w, b.reshape(1, _O), g.reshape(1, _O), beta.reshape(1, _O),
      qmat)


def _score_body(aq, ei, ai, et, oqi, oit):
    nt = (((1,), (1,)), ((), ()))
    oqi[...] = lax.dot_general(aq[...], ei[...], nt,
                               preferred_element_type=jnp.float32)
    oit[...] = lax.dot_general(ai[...], et[...], nt,
                               preferred_element_type=jnp.float32)


def _scores(aq, ei, ai, et):
    nb = 50
    blk = _N // nb
    return pl.pallas_call(
        _score_body,
        grid=(nb,),
        in_specs=[
            pl.BlockSpec((blk, _O), lambda b: (b, 0)),
            pl.BlockSpec((_N, _O), lambda b: (0, 0)),
            pl.BlockSpec((blk, _O), lambda b: (b, 0)),
            pl.BlockSpec((_N, _O), lambda b: (0, 0)),
        ],
        out_specs=[
            pl.BlockSpec((blk, _N), lambda b: (b, 0)),
            pl.BlockSpec((blk, _N), lambda b: (b, 0)),
        ],
        out_shape=[jax.ShapeDtypeStruct((_N, _N), jnp.float32)] * 2,
    )(aq, ei, ai, et)


def kernel(feature_q, feature_i, feature_t, qi_row, qi_col, qi_val,
           it_row, it_col, it_val, W, W_q, b_q, g_q, beta_q,
           W_i, b_i, g_i, beta_i, W_t, b_t, g_t, beta_t, Q):
    w_split = W.reshape(_D, 2, _HALF).transpose(1, 0, 2)
    xq = _project(feature_q, w_split)
    xi = _project(feature_i, w_split)
    xt = _project(feature_t, w_split)

    i32 = jnp.int32
    acc_q, acc_i, acc_t = _spmm(
        xq, xi, xt,
        qi_row.astype(i32), qi_col.astype(i32),
        it_row.astype(i32), it_col.astype(i32),
        qi_val, it_val,
    )

    _, aq = _embed(acc_q, feature_q, W_q, b_q, g_q, beta_q, Q)
    ei, ai = _embed(acc_i, feature_i, W_i, b_i, g_i, beta_i, Q)
    et, _ = _embed(acc_t, feature_t, W_t, b_t, g_t, beta_t, Q)
    score_qi, score_it = _scores(aq, ei, ai, et)
    return (score_qi, score_it)
